# Initial kernel scaffold; baseline (speedup 1.0000x reference)
import jax
import jax.numpy as jnp
from jax.experimental import pallas as pl


def kernel(x, r, edge_index, Wq, Wk, Wv, Wkr, Wvr, Wout, Wg, bg, Ws, bs, W1, b1, W2, b2, ln1_g, ln1_b, lnr_g, lnr_b, post_g, post_b, ffpre_g, ffpre_b, ffpost_g, ffpost_b):
    raise NotImplementedError("write your pallas kernel here")


# speedup vs baseline: 41.9323x; 41.9323x over previous
import functools

import jax
import jax.numpy as jnp
from jax import lax
from jax.experimental import pallas as pl
from jax.experimental.pallas import tpu as pltpu
from jax.experimental.pallas import tpu_sc as plsc

N = 10000
E = 160000
D = 128
NH = 16
DH = 8
F = 4 * D

_EPS = 1e-5


def _layernorm(x, g, b):
    mu = jnp.mean(x, axis=-1, keepdims=True)
    var = jnp.mean((x - mu) ** 2, axis=-1, keepdims=True)
    return (x - mu) / jnp.sqrt(var + _EPS) * g + b


def _pre_node_body(x_ref, wq_ref, wk_ref, wv_ref, g_ref, b_ref,
                   xn_ref, pq_ref, k_ref, v_ref):
    xn = _layernorm(x_ref[...], g_ref[...], b_ref[...])
    xn_ref[...] = xn
    q = jnp.dot(xn, wq_ref[...], preferred_element_type=jnp.float32)
    nb = x_ref.shape[0]
    rows = lax.broadcasted_iota(jnp.int32, (nb, NH), 0) + pl.program_id(0) * nb
    slot = jnp.bitwise_and(rows, DH - 1).astype(jnp.float32)
    owner = (rows >= (N // 2)).astype(jnp.float32)
    pad = jnp.zeros((nb, D - 2 * NH), jnp.float32)
    pq_ref[...] = jnp.concatenate(
        [jnp.where(q > 0, q + 1.0, jnp.exp(q)), slot, owner, pad], axis=-1)
    k_ref[...] = jnp.dot(xn, wk_ref[...], preferred_element_type=jnp.float32)
    v_ref[...] = jnp.dot(xn, wv_ref[...], preferred_element_type=jnp.float32)


def _pre_edge_body(r_ref, wkr_ref, wvr_ref, g_ref, b_ref, kr_ref, vr_ref):
    rn = _layernorm(r_ref[...], g_ref[...], b_ref[...])
    kr_ref[...] = jnp.dot(rn, wkr_ref[...], preferred_element_type=jnp.float32)
    vr_ref[...] = jnp.dot(rn, wvr_ref[...], preferred_element_type=jnp.float32)


def _post_body(x_ref, xn_ref, num_ref, den_ref, wout_ref, wg1_ref, wg2_ref,
               bg_ref, ws_ref, bs_ref, w1_ref, b1_ref, w2_ref, b2_ref,
               postg_ref, postb_ref, ffpreg_ref, ffpreb_ref, ffpostg_ref,
               ffpostb_ref, y_ref):
    num = num_ref[...]
    den = jnp.maximum(den_ref[...], 1e-6)
    den_full = jnp.concatenate([den] * DH, axis=-1)
    attn = num / den_full
    out = jnp.dot(attn, wout_ref[...], preferred_element_type=jnp.float32)
    xn = xn_ref[...]
    gate_pre = (jnp.dot(out, wg1_ref[...], preferred_element_type=jnp.float32)
                + jnp.dot(xn, wg2_ref[...], preferred_element_type=jnp.float32)
                + bg_ref[...])
    g = jax.nn.sigmoid(gate_pre)
    skip = jnp.dot(xn, ws_ref[...], preferred_element_type=jnp.float32) + bs_ref[...]
    out = out + g * (skip - out)
    x_mid = x_ref[...] + _layernorm(out, postg_ref[...], postb_ref[...])
    h = _layernorm(x_mid, ffpreg_ref[...], ffpreb_ref[...])
    h = jnp.dot(h, w1_ref[...], preferred_element_type=jnp.float32) + b1_ref[...]
    h = 0.5 * h * (1.0 + lax.erf(h * (2.0 ** -0.5)))
    h = jnp.dot(h, w2_ref[...], preferred_element_type=jnp.float32) + b2_ref[...]
    y_ref[...] = x_mid + _layernorm(h, ffpostg_ref[...], ffpostb_ref[...])


_C = 64
_NCHUNK = E // _C
_NC = 2
_NS = 16
_CHUNKS_PER_T = -(-_NCHUNK // _NS)
_L = 16
_HALF = N // _NC
_NUMROWS = 5008
_DUMPQ = _HALF // DH
_DROWS = 632
_TROWS = 320
_TROWS_TAIL = _HALF - 15 * _TROWS


def _edge_sc_body(src_hbm, dst_hbm, pq_hbm, k_hbm, v_hbm, kr_hbm, vr_hbm,
                  num_out, den_out,
                  src_v, dst_v, rownum_v, rowden_v, krow, vrow, qrow,
                  krv, vrv, numv, denv, denu, num_sh, den_sh, sem):
    cid = lax.axis_index("c")
    sid = lax.axis_index("s")
    last = sid == _NS - 1
    cidf = lax.convert_element_type(cid, jnp.float32)
    half0 = cid * _HALF

    def _zero_row(i, carry):
        for j in range(DH):
            numv[i, pl.ds(j * _L, _L)] = jnp.zeros((_L,), jnp.float32)
        return carry

    lax.fori_loop(0, _C, _zero_row, 0)
    nzcopy = jnp.where(last, 3, 5)

    def _zero_num(i, carry):
        pltpu.sync_copy(numv, num_sh.at[pl.ds(sid * _TROWS + i * _C, _C)])
        return carry

    lax.fori_loop(0, nzcopy, _zero_num, 0)

    @pl.when(jnp.logical_not(last))
    def _zero_rest():
        pltpu.sync_copy(numv.at[pl.ds(0, 40)],
                        den_sh.at[pl.ds(sid * 40, 40)])

    @pl.when(last)
    def _zero_rest_tail():
        pltpu.sync_copy(numv.at[pl.ds(0, 16)],
                        num_sh.at[pl.ds(15 * _TROWS + 192, 16)])
        pltpu.sync_copy(numv.at[pl.ds(0, 32)],
                        den_sh.at[pl.ds(600, 32)])

    plsc.subcore_barrier()

    def _chunk(j, carry):
        ci = j * _NS + sid

        @pl.when(ci < _NCHUNK)
        def _run():
            _do_chunk(ci)
        return carry

    def _do_chunk(ci):
        base = ci * _C
        pltpu.sync_copy(src_hbm.at[pl.ds(base, _C)], src_v)
        pltpu.sync_copy(dst_hbm.at[pl.ds(base, _C)], dst_v)
        cps = (pltpu.async_copy(k_hbm.at[src_v], krow, sem),
               pltpu.async_copy(v_hbm.at[src_v], vrow, sem),
               pltpu.async_copy(pq_hbm.at[dst_v], qrow, sem),
               pltpu.async_copy(kr_hbm.at[pl.ds(base, _C)], krv, sem),
               pltpu.async_copy(vr_hbm.at[pl.ds(base, _C)], vrv, sem))
        for cp in cps:
            cp.wait()

        def _xform(i, carry2):
            sl = pl.ds(i * _L, _L)
            dloc = dst_v[sl] - half0
            m = jnp.logical_and(dloc >= 0, dloc < _HALF)
            rownum_v[sl] = jnp.where(m, dloc, _HALF)
            rowden_v[sl] = jnp.where(
                m, lax.shift_right_logical(dloc, 3), _DUMPQ)
            return carry2

        lax.fori_loop(0, _C // _L, _xform, 0)

        def _edge(e, ecarry):
            a = jnp.zeros((_L,), jnp.float32)
            for j8 in range(DH):
                sl = pl.ds(j8 * _L, _L)
                ke = krow[e, sl] + krv[e, sl]
                pk = jnp.where(ke > 0, ke + 1.0, jnp.exp(ke))
                a = a + qrow[e, sl] * pk
            for j8 in range(DH):
                sl = pl.ds(j8 * _L, _L)
                numv[e, sl] = a * (vrow[e, sl] + vrv[e, sl])
            s8 = qrow[e, pl.ds(D, _L)]
            zero = jnp.zeros((_L,), jnp.float32)
            for j8 in range(DH):
                denv[e, pl.ds(j8 * _L, _L)] = jnp.where(
                    s8 == jnp.float32(j8), a, zero)
            return ecarry

        lax.fori_loop(0, _C, _edge, 0)
        pltpu.sync_copy(numv, num_sh.at[rownum_v], add=True)
        pltpu.sync_copy(denv, den_sh.at[rowden_v], add=True)

    lax.fori_loop(0, _CHUNKS_PER_T, _chunk, 0)
    plsc.subcore_barrier()

    row0 = sid * _TROWS

    @pl.when(jnp.logical_not(last))
    def _out_main():
        pltpu.sync_copy(num_sh.at[pl.ds(row0, _TROWS)],
                        num_out.at[pl.ds(half0 + row0, _TROWS)])
        pltpu.sync_copy(den_sh.at[pl.ds(sid * (_TROWS // DH), _TROWS // DH)],
                        numv.at[pl.ds(0, _TROWS // DH)])

    @pl.when(last)
    def _out_tail():
        pltpu.sync_copy(num_sh.at[pl.ds(row0, _TROWS_TAIL)],
                        num_out.at[pl.ds(half0 + row0, _TROWS_TAIL)])
        pltpu.sync_copy(den_sh.at[pl.ds(15 * (_TROWS // DH), _TROWS_TAIL // DH)],
                        numv.at[pl.ds(0, _TROWS_TAIL // DH)])

    ngrp = jnp.where(last, _TROWS_TAIL // 40, _TROWS // 40)

    def _den_group(c, carry):
        def _unpack(i, carry2):
            n = c * 40 + i
            q8, s = n // DH, (n % DH) * _L
            denu[i, :] = numv[q8, pl.ds(s, _L)]
            return carry2

        lax.fori_loop(0, 40, _unpack, 0)
        pltpu.sync_copy(denu, den_out.at[pl.ds(half0 + row0 + c * 40, 40)])
        return carry

    lax.fori_loop(0, ngrp, _den_group, 0)


@functools.cache
def _build_edge_sc():
    return functools.partial(
        pl.kernel,
        out_type=(jax.ShapeDtypeStruct((N, D), jnp.float32),
                  jax.ShapeDtypeStruct((N, NH), jnp.float32)),
        mesh=plsc.VectorSubcoreMesh(core_axis_name="c", subcore_axis_name="s",
                                    num_cores=_NC, num_subcores=_NS),
        scratch_types=[
            pltpu.VMEM((_C,), jnp.int32),
            pltpu.VMEM((_C,), jnp.int32),
            pltpu.VMEM((_C,), jnp.int32),
            pltpu.VMEM((_C,), jnp.int32),
            pltpu.VMEM((_C, D), jnp.float32),
            pltpu.VMEM((_C, D), jnp.float32),
            pltpu.VMEM((_C, 2 * D), jnp.float32),
            pltpu.VMEM((_C, D), jnp.float32),
            pltpu.VMEM((_C, D), jnp.float32),
            pltpu.VMEM((_C, D), jnp.float32),
            pltpu.VMEM((_C, D), jnp.float32),
            pltpu.VMEM((40, NH), jnp.float32),
            pltpu.VMEM_SHARED((_NUMROWS, D), jnp.float32),
            pltpu.VMEM_SHARED((_DROWS, D), jnp.float32),
            pltpu.SemaphoreType.DMA,
        ],
    )(_edge_sc_body)


_BN = 2000
_BE = 4000


def _rep_spec(shape):
    return pl.BlockSpec(shape, lambda i: (0,) * len(shape))


def kernel(x, r, edge_index, Wq, Wk, Wv, Wkr, Wvr, Wout, Wg, bg, Ws, bs,
           W1, b1, W2, b2, ln1_g, ln1_b, lnr_g, lnr_b, post_g, post_b,
           ffpre_g, ffpre_b, ffpost_g, ffpost_b):
    src = edge_index[0].astype(jnp.int32)
    dst = edge_index[1].astype(jnp.int32)

    def perm_cols(w):
        return w.reshape(D, NH, DH).transpose(0, 2, 1).reshape(D, NH * DH)

    wq_t = perm_cols(Wq)
    wk_t = perm_cols(Wk)
    wv_t = perm_cols(Wv)
    wkr_t = perm_cols(Wkr)
    wvr_t = perm_cols(Wvr)
    wout_p = Wout.reshape(NH, DH, D).transpose(1, 0, 2).reshape(NH * DH, D)
    wg1 = Wg[:D]
    wg2 = Wg[D:]

    def row(v):
        return v.reshape(1, -1)

    w_spec = _rep_spec((D, D))
    g_spec = _rep_spec((1, D))

    xn, pq, k, v = pl.pallas_call(
        _pre_node_body,
        grid=(N // _BN,),
        in_specs=[pl.BlockSpec((_BN, D), lambda i: (i, 0)),
                  w_spec, w_spec, w_spec, g_spec, g_spec],
        out_specs=[pl.BlockSpec((_BN, D), lambda i: (i, 0)),
                   pl.BlockSpec((_BN, 2 * D), lambda i: (i, 0)),
                   pl.BlockSpec((_BN, D), lambda i: (i, 0)),
                   pl.BlockSpec((_BN, D), lambda i: (i, 0))],
        out_shape=[jax.ShapeDtypeStruct((N, D), jnp.float32),
                   jax.ShapeDtypeStruct((N, 2 * D), jnp.float32),
                   jax.ShapeDtypeStruct((N, D), jnp.float32),
                   jax.ShapeDtypeStruct((N, D), jnp.float32)],
    )(x, wq_t, wk_t, wv_t, row(ln1_g), row(ln1_b))

    kr, vr = pl.pallas_call(
        _pre_edge_body,
        grid=(E // _BE,),
        in_specs=[pl.BlockSpec((_BE, D), lambda i: (i, 0)),
                  w_spec, w_spec, g_spec, g_spec],
        out_specs=[pl.BlockSpec((_BE, D), lambda i: (i, 0))] * 2,
        out_shape=[jax.ShapeDtypeStruct((E, D), jnp.float32)] * 2,
    )(r, wkr_t, wvr_t, row(lnr_g), row(lnr_b))

    num, den = _build_edge_sc()(src, dst, pq, k, v, kr, vr)

    y = pl.pallas_call(
        _post_body,
        grid=(N // _BN,),
        in_specs=[pl.BlockSpec((_BN, D), lambda i: (i, 0)),
                  pl.BlockSpec((_BN, D), lambda i: (i, 0)),
                  pl.BlockSpec((_BN, D), lambda i: (i, 0)),
                  pl.BlockSpec((_BN, NH), lambda i: (i, 0)),
                  w_spec, w_spec, w_spec, g_spec, w_spec, g_spec,
                  _rep_spec((D, F)), _rep_spec((1, F)),
                  _rep_spec((F, D)), g_spec,
                  g_spec, g_spec, g_spec, g_spec, g_spec, g_spec],
        out_specs=pl.BlockSpec((_BN, D), lambda i: (i, 0)),
        out_shape=jax.ShapeDtypeStruct((N, D), jnp.float32),
    )(x, xn, num, den, wout_p, wg1, wg2, row(bg), Ws, row(bs),
      W1, row(b1), W2, row(b2), row(post_g), row(post_b),
      row(ffpre_g), row(ffpre_b), row(ffpost_g), row(ffpost_b))

    return y


# speedup vs baseline: 46.0241x; 1.0976x over previous
import functools

import jax
import jax.numpy as jnp
from jax import lax
from jax.experimental import pallas as pl
from jax.experimental.pallas import tpu as pltpu
from jax.experimental.pallas import tpu_sc as plsc

N = 10000
E = 160000
D = 128
NH = 16
DH = 8
F = 4 * D

_EPS = 1e-5


def _layernorm(x, g, b):
    mu = jnp.mean(x, axis=-1, keepdims=True)
    var = jnp.mean((x - mu) ** 2, axis=-1, keepdims=True)
    return (x - mu) / jnp.sqrt(var + _EPS) * g + b


def _pre_node_body(x_ref, wq_ref, wk_ref, wv_ref, g_ref, b_ref,
                   xn_ref, pq_ref, kv_ref):
    xn = _layernorm(x_ref[...], g_ref[...], b_ref[...])
    xn_ref[...] = xn
    q = jnp.dot(xn, wq_ref[...], preferred_element_type=jnp.float32)
    pq = jnp.where(q > 0, q + 1.0, jnp.exp(q))
    nb = x_ref.shape[0]
    rows = lax.broadcasted_iota(jnp.int32, (nb, NH), 0) + pl.program_id(0) * nb
    slot = jnp.bitwise_and(rows, DH - 1).astype(jnp.float32)
    pad = jnp.zeros((nb, D - NH), jnp.float32)
    pq_ref[...] = jnp.concatenate([pq, slot, pad], axis=-1)
    k = jnp.dot(xn, wk_ref[...], preferred_element_type=jnp.float32)
    v = jnp.dot(xn, wv_ref[...], preferred_element_type=jnp.float32)
    kv_ref[...] = jnp.concatenate([k, v], axis=-1)


def _pre_edge_body(r_ref, wkr_ref, wvr_ref, g_ref, b_ref, kvr_ref):
    rn = _layernorm(r_ref[...], g_ref[...], b_ref[...])
    kr = jnp.dot(rn, wkr_ref[...], preferred_element_type=jnp.float32)
    vr = jnp.dot(rn, wvr_ref[...], preferred_element_type=jnp.float32)
    kvr_ref[...] = jnp.concatenate([kr, vr], axis=-1)


def _post_body(x_ref, xn_ref, num_ref, den_ref, wout_ref, wg1_ref, wg2_ref,
               bg_ref, ws_ref, bs_ref, w1_ref, b1_ref, w2_ref, b2_ref,
               postg_ref, postb_ref, ffpreg_ref, ffpreb_ref, ffpostg_ref,
               ffpostb_ref, y_ref):
    num = num_ref[0] + num_ref[1]
    den = jnp.maximum(den_ref[0] + den_ref[1], 1e-6)
    den_full = jnp.concatenate([den] * DH, axis=-1)
    attn = num / den_full
    out = jnp.dot(attn, wout_ref[...], preferred_element_type=jnp.float32)
    xn = xn_ref[...]
    gate_pre = (jnp.dot(out, wg1_ref[...], preferred_element_type=jnp.float32)
                + jnp.dot(xn, wg2_ref[...], preferred_element_type=jnp.float32)
                + bg_ref[...])
    g = jax.nn.sigmoid(gate_pre)
    skip = jnp.dot(xn, ws_ref[...], preferred_element_type=jnp.float32) + bs_ref[...]
    out = out + g * (skip - out)
    x_mid = x_ref[...] + _layernorm(out, postg_ref[...], postb_ref[...])
    h = _layernorm(x_mid, ffpreg_ref[...], ffpreb_ref[...])
    h = jnp.dot(h, w1_ref[...], preferred_element_type=jnp.float32) + b1_ref[...]
    h = 0.5 * h * (1.0 + lax.erf(h * (2.0 ** -0.5)))
    h = jnp.dot(h, w2_ref[...], preferred_element_type=jnp.float32) + b2_ref[...]
    y_ref[...] = x_mid + _layernorm(h, ffpostg_ref[...], ffpostb_ref[...])


_C = 32
_NCHUNK = E // _C
_NC = 2
_NS = 16
_NW = _NC * _NS
_CHUNKS_PER_W = -(-_NCHUNK // _NW)
_L = 16
_NUMROWS = 10240
_DROWS = 1280
_TROWS = 640
_OUT_TAIL = N - 15 * _TROWS


def _edge_sc_body(src_hbm, dst_hbm, pq_hbm, kv_hbm, kvr_hbm,
                  num_out, den_out,
                  src_v, dst_v, rowden_v, kvrow, qrow, kvr, numv, denv, denu,
                  num_sh, den_sh, sem):
    cid = lax.axis_index("c")
    sid = lax.axis_index("s")
    wid = sid * _NC + cid
    last = sid == _NS - 1

    def _zero_row(i, carry):
        for j in range(DH):
            numv[i, pl.ds(j * _L, _L)] = jnp.zeros((_L,), jnp.float32)
        return carry

    lax.fori_loop(0, _C, _zero_row, 0)

    def _zero_num(i, carry):
        pltpu.sync_copy(numv, num_sh.at[pl.ds(sid * _TROWS + i * _C, _C)])
        return carry

    lax.fori_loop(0, _TROWS // _C, _zero_num, 0)
    pltpu.sync_copy(numv, den_sh.at[pl.ds(sid * 80, _C)])
    pltpu.sync_copy(numv, den_sh.at[pl.ds(sid * 80 + _C, _C)])
    pltpu.sync_copy(numv.at[pl.ds(0, 16)], den_sh.at[pl.ds(sid * 80 + 64, 16)])
    plsc.subcore_barrier()

    def _chunk(j, carry):
        ci = j * _NW + wid

        @pl.when(ci < _NCHUNK)
        def _run():
            _do_chunk(ci)

        return carry

    def _do_chunk(ci):
        base = ci * _C
        pltpu.sync_copy(src_hbm.at[pl.ds(base, _C)], src_v)
        pltpu.sync_copy(dst_hbm.at[pl.ds(base, _C)], dst_v)
        cps = (pltpu.async_copy(kv_hbm.at[src_v], kvrow, sem),
               pltpu.async_copy(pq_hbm.at[dst_v], qrow, sem),
               pltpu.async_copy(kvr_hbm.at[pl.ds(base, _C)], kvr, sem))
        for cp in cps:
            cp.wait()

        def _xform(i, carry2):
            sl = pl.ds(i * _L, _L)
            rowden_v[sl] = lax.shift_right_logical(dst_v[sl], 3)
            return carry2

        lax.fori_loop(0, _C // _L, _xform, 0)

        def _edge(e, ecarry):
            s8 = qrow[e, pl.ds(D, _L)]
            a = jnp.zeros((_L,), jnp.float32)
            for j8 in range(DH):
                sl = pl.ds(j8 * _L, _L)
                ke = kvrow[e, sl] + kvr[e, sl]
                pk = jnp.where(ke > 0, ke + 1.0, jnp.exp(ke))
                a = a + qrow[e, sl] * pk
            for j8 in range(DH):
                numv[e, pl.ds(j8 * _L, _L)] = a * (
                    kvrow[e, pl.ds(D + j8 * _L, _L)]
                    + kvr[e, pl.ds(D + j8 * _L, _L)])
            zero = jnp.zeros((_L,), jnp.float32)
            for j8 in range(DH):
                denv[e, pl.ds(j8 * _L, _L)] = jnp.where(
                    s8 == jnp.float32(j8), a, zero)
            return ecarry

        lax.fori_loop(0, _C, _edge, 0)
        pltpu.sync_copy(numv, num_sh.at[dst_v], add=True)
        pltpu.sync_copy(denv, den_sh.at[rowden_v], add=True)

    lax.fori_loop(0, _CHUNKS_PER_W, _chunk, 0)
    plsc.subcore_barrier()

    row0 = sid * _TROWS

    @pl.when(jnp.logical_not(last))
    def _out_main():
        pltpu.sync_copy(num_sh.at[pl.ds(row0, _TROWS)],
                        num_out.at[cid, pl.ds(row0, _TROWS)])

    @pl.when(last)
    def _out_tail():
        pltpu.sync_copy(num_sh.at[pl.ds(row0, _OUT_TAIL)],
                        num_out.at[cid, pl.ds(row0, _OUT_TAIL)])

    ngrp = jnp.where(last, _OUT_TAIL // 64, _TROWS // 64)

    def _den_group(c, carry):
        pltpu.sync_copy(den_sh.at[pl.ds(sid * 80 + c * 8, 8)],
                        numv.at[pl.ds(0, 8)])
        for hlf in range(2):
            def _unpack(i, carry2):
                n = hlf * 32 + i
                denu[i, :] = numv[n // DH, pl.ds((n % DH) * _L, _L)]
                return carry2

            lax.fori_loop(0, 32, _unpack, 0)
            pltpu.sync_copy(
                denu, den_out.at[cid, pl.ds(row0 + c * 64 + hlf * 32, 32)])
        return carry

    lax.fori_loop(0, ngrp, _den_group, 0)

    @pl.when(last)
    def _den_tail():
        pltpu.sync_copy(den_sh.at[pl.ds(sid * 80 + 48, 8)], numv.at[pl.ds(0, 8)])

        def _unpack(i, carry2):
            denu[i, :] = numv[i // DH, pl.ds((i % DH) * _L, _L)]
            return carry2

        lax.fori_loop(0, 16, _unpack, 0)
        pltpu.sync_copy(denu.at[pl.ds(0, 16)],
                        den_out.at[cid, pl.ds(9984, 16)])


@functools.cache
def _build_edge_sc():
    return functools.partial(
        pl.kernel,
        out_type=(jax.ShapeDtypeStruct((_NC, N, D), jnp.float32),
                  jax.ShapeDtypeStruct((_NC, N, NH), jnp.float32)),
        mesh=plsc.VectorSubcoreMesh(core_axis_name="c", subcore_axis_name="s",
                                    num_cores=_NC, num_subcores=_NS),
        scratch_types=[
            pltpu.VMEM((_C,), jnp.int32),
            pltpu.VMEM((_C,), jnp.int32),
            pltpu.VMEM((_C,), jnp.int32),
            pltpu.VMEM((_C, 2 * D), jnp.float32),
            pltpu.VMEM((_C, 2 * D), jnp.float32),
            pltpu.VMEM((_C, 2 * D), jnp.float32),
            pltpu.VMEM((_C, D), jnp.float32),
            pltpu.VMEM((_C, D), jnp.float32),
            pltpu.VMEM((32, NH), jnp.float32),
            pltpu.VMEM_SHARED((_NUMROWS, D), jnp.float32),
            pltpu.VMEM_SHARED((_DROWS, D), jnp.float32),
            pltpu.SemaphoreType.DMA,
        ],
    )(_edge_sc_body)


_BN = 2000
_BE = 4000


def _rep_spec(shape):
    return pl.BlockSpec(shape, lambda i: (0,) * len(shape))


def kernel(x, r, edge_index, Wq, Wk, Wv, Wkr, Wvr, Wout, Wg, bg, Ws, bs,
           W1, b1, W2, b2, ln1_g, ln1_b, lnr_g, lnr_b, post_g, post_b,
           ffpre_g, ffpre_b, ffpost_g, ffpost_b):
    src = edge_index[0].astype(jnp.int32)
    dst = edge_index[1].astype(jnp.int32)

    def perm_cols(w):
        return w.reshape(D, NH, DH).transpose(0, 2, 1).reshape(D, NH * DH)

    wq_t = perm_cols(Wq)
    wk_t = perm_cols(Wk)
    wv_t = perm_cols(Wv)
    wkr_t = perm_cols(Wkr)
    wvr_t = perm_cols(Wvr)
    wout_p = Wout.reshape(NH, DH, D).transpose(1, 0, 2).reshape(NH * DH, D)
    wg1 = Wg[:D]
    wg2 = Wg[D:]

    def row(v):
        return v.reshape(1, -1)

    w_spec = _rep_spec((D, D))
    g_spec = _rep_spec((1, D))

    xn, pq, kv = pl.pallas_call(
        _pre_node_body,
        grid=(N // _BN,),
        in_specs=[pl.BlockSpec((_BN, D), lambda i: (i, 0)),
                  w_spec, w_spec, w_spec, g_spec, g_spec],
        out_specs=[pl.BlockSpec((_BN, D), lambda i: (i, 0)),
                   pl.BlockSpec((_BN, 2 * D), lambda i: (i, 0)),
                   pl.BlockSpec((_BN, 2 * D), lambda i: (i, 0))],
        out_shape=[jax.ShapeDtypeStruct((N, D), jnp.float32),
                   jax.ShapeDtypeStruct((N, 2 * D), jnp.float32),
                   jax.ShapeDtypeStruct((N, 2 * D), jnp.float32)],
    )(x, wq_t, wk_t, wv_t, row(ln1_g), row(ln1_b))

    kvr = pl.pallas_call(
        _pre_edge_body,
        grid=(E // _BE,),
        in_specs=[pl.BlockSpec((_BE, D), lambda i: (i, 0)),
                  w_spec, w_spec, g_spec, g_spec],
        out_specs=pl.BlockSpec((_BE, 2 * D), lambda i: (i, 0)),
        out_shape=jax.ShapeDtypeStruct((E, 2 * D), jnp.float32),
    )(r, wkr_t, wvr_t, row(lnr_g), row(lnr_b))

    num_p, den_p = _build_edge_sc()(src, dst, pq, kv, kvr)

    y = pl.pallas_call(
        _post_body,
        grid=(N // _BN,),
        in_specs=[pl.BlockSpec((_BN, D), lambda i: (i, 0)),
                  pl.BlockSpec((_BN, D), lambda i: (i, 0)),
                  pl.BlockSpec((_NC, _BN, D), lambda i: (0, i, 0)),
                  pl.BlockSpec((_NC, _BN, NH), lambda i: (0, i, 0)),
                  w_spec, w_spec, w_spec, g_spec, w_spec, g_spec,
                  _rep_spec((D, F)), _rep_spec((1, F)),
                  _rep_spec((F, D)), g_spec,
                  g_spec, g_spec, g_spec, g_spec, g_spec, g_spec],
        out_specs=pl.BlockSpec((_BN, D), lambda i: (i, 0)),
        out_shape=jax.ShapeDtypeStruct((N, D), jnp.float32),
    )(x, xn, num_p, den_p, wout_p, wg1, wg2, row(bg), Ws, row(bs),
      W1, row(b1), W2, row(b2), row(post_g), row(post_b),
      row(ffpre_g), row(ffpre_b), row(ffpost_g), row(ffpost_b))

    return y


# speedup vs baseline: 80.2489x; 1.7436x over previous
import functools

import jax
import jax.numpy as jnp
from jax import lax
from jax.experimental import pallas as pl
from jax.experimental.pallas import tpu as pltpu
from jax.experimental.pallas import tpu_sc as plsc

N = 10000
E = 160000
D = 128
NH = 16
DH = 8
F = 4 * D

_EPS = 1e-5


def _layernorm(x, g, b):
    mu = jnp.mean(x, axis=-1, keepdims=True)
    var = jnp.mean((x - mu) ** 2, axis=-1, keepdims=True)
    return (x - mu) / jnp.sqrt(var + _EPS) * g + b


def _pre_node_body(x_ref, wq_ref, wk_ref, wv_ref, g_ref, b_ref,
                   xn_ref, pq_ref, kv_ref):
    xn = _layernorm(x_ref[...], g_ref[...], b_ref[...])
    xn_ref[...] = xn
    q = jnp.dot(xn, wq_ref[...], preferred_element_type=jnp.float32)
    pq = jnp.where(q > 0, q + 1.0, jnp.exp(q))
    nb = x_ref.shape[0]
    rows = lax.broadcasted_iota(jnp.int32, (nb, NH), 0) + pl.program_id(0) * nb
    slot = jnp.bitwise_and(rows, DH - 1).astype(jnp.float32)
    pad = jnp.zeros((nb, D - NH), jnp.float32)
    pq_ref[...] = jnp.concatenate([pq, slot, pad], axis=-1)
    k = jnp.dot(xn, wk_ref[...], preferred_element_type=jnp.float32)
    v = jnp.dot(xn, wv_ref[...], preferred_element_type=jnp.float32)
    kv_ref[...] = jnp.concatenate([k, v], axis=-1)


def _pre_edge_body(r_ref, wkr_ref, wvr_ref, g_ref, b_ref, kvr_ref):
    rn = _layernorm(r_ref[...], g_ref[...], b_ref[...])
    kr = jnp.dot(rn, wkr_ref[...], preferred_element_type=jnp.float32)
    vr = jnp.dot(rn, wvr_ref[...], preferred_element_type=jnp.float32)
    kvr_ref[...] = jnp.concatenate([kr, vr], axis=-1)


def _post_body(x_ref, xn_ref, num_ref, den_ref, wout_ref, wg1_ref, wg2_ref,
               bg_ref, ws_ref, bs_ref, w1_ref, b1_ref, w2_ref, b2_ref,
               postg_ref, postb_ref, ffpreg_ref, ffpreb_ref, ffpostg_ref,
               ffpostb_ref, y_ref):
    num = num_ref[0] + num_ref[1]
    den = jnp.maximum(den_ref[0] + den_ref[1], 1e-6)
    den_full = jnp.concatenate([den] * DH, axis=-1)
    attn = num / den_full
    out = jnp.dot(attn, wout_ref[...], preferred_element_type=jnp.float32)
    xn = xn_ref[...]
    gate_pre = (jnp.dot(out, wg1_ref[...], preferred_element_type=jnp.float32)
                + jnp.dot(xn, wg2_ref[...], preferred_element_type=jnp.float32)
                + bg_ref[...])
    g = jax.nn.sigmoid(gate_pre)
    skip = jnp.dot(xn, ws_ref[...], preferred_element_type=jnp.float32) + bs_ref[...]
    out = out + g * (skip - out)
    x_mid = x_ref[...] + _layernorm(out, postg_ref[...], postb_ref[...])
    h = _layernorm(x_mid, ffpreg_ref[...], ffpreb_ref[...])
    h = jnp.dot(h, w1_ref[...], preferred_element_type=jnp.float32) + b1_ref[...]
    h = 0.5 * h * (1.0 + lax.erf(h * (2.0 ** -0.5)))
    h = jnp.dot(h, w2_ref[...], preferred_element_type=jnp.float32) + b2_ref[...]
    y_ref[...] = x_mid + _layernorm(h, ffpostg_ref[...], ffpostb_ref[...])


_C = 16
_NCHUNK = E // _C
_NC = 2
_NS = 16
_NW = _NC * _NS
_L = 16
_NUMROWS = 10240
_DROWS = 1280
_ACCROWS = _NUMROWS + _DROWS
_TROWS = 640
_OUT_TAIL = N - 15 * _TROWS
_ZROWS = _ACCROWS // _NS


def _edge_sc_body(sd_hbm, kvpq_hbm, kvr_hbm, num_out, den_out,
                  sd0, sd1, gidx0, gidx1, sidx0, sidx1,
                  grow0, grow1, kvr0, kvr1, sval0, sval1, denu,
                  acc_sh, sem_i, sem_g, sem_s):
    cid = lax.axis_index("c")
    sid = lax.axis_index("s")
    wid = sid * _NC + cid
    last = sid == _NS - 1
    sd = (sd0, sd1)
    gidx = (gidx0, gidx1)
    sidx = (sidx0, sidx1)
    grow = (grow0, grow1)
    kvrb = (kvr0, kvr1)
    sval = (sval0, sval1)
    nj = (_NCHUNK - wid + _NW - 1) // _NW

    def _zero_row(i, carry):
        for j in range(DH):
            sval0[i, pl.ds(j * _L, _L)] = jnp.zeros((_L,), jnp.float32)
        return carry

    lax.fori_loop(0, 2 * _C, _zero_row, 0)
    z0 = sid * _ZROWS

    def _zero_acc(i, carry):
        pltpu.sync_copy(sval0, acc_sh.at[pl.ds(z0 + i * 32, 32)])
        return carry

    lax.fori_loop(0, _ZROWS // 32, _zero_acc, 0)
    pltpu.sync_copy(sval0.at[pl.ds(0, _ZROWS % 32)],
                    acc_sh.at[pl.ds(z0 + (_ZROWS // 32) * 32, _ZROWS % 32)])
    plsc.subcore_barrier()

    def _build_gidx(p):
        sdv = sd[p][...]
        gidx[p][pl.ds(0, _L)] = lax.shift_right_logical(sdv, 14)
        gidx[p][pl.ds(_L, _L)] = N + jnp.bitwise_and(sdv, 16383)

    def _build_sidx(p):
        sdv = sd[p][...]
        dstv = jnp.bitwise_and(sdv, 16383)
        sidx[p][pl.ds(0, _L)] = dstv
        sidx[p][pl.ds(_L, _L)] = _NUMROWS + lax.shift_right_logical(dstv, 3)

    def _issue_gathers(p, ci):
        pltpu.async_copy(kvpq_hbm.at[gidx[p]], grow[p], sem_g)
        pltpu.async_copy(kvr_hbm.at[pl.ds(ci * _C, _C)], kvrb[p], sem_g)

    def _drain_gathers(p, ci):
        pltpu.make_async_copy(kvpq_hbm.at[gidx[p]], grow[p], sem_g).wait()
        pltpu.make_async_copy(kvr_hbm.at[pl.ds(ci * _C, _C)], kvrb[p],
                              sem_g).wait()

    def _compute(p):
        g = grow[p]
        kv_r = kvrb[p]
        sv = sval[p]

        def _edge(e, ecarry):
            s8 = g[_C + e, pl.ds(D, _L)]
            a = jnp.zeros((_L,), jnp.float32)
            for j8 in range(DH):
                sl = pl.ds(j8 * _L, _L)
                ke = g[e, sl] + kv_r[e, sl]
                pk = jnp.where(ke > 0, ke + 1.0, jnp.exp(ke))
                a = a + g[_C + e, sl] * pk
            for j8 in range(DH):
                sl = pl.ds(D + j8 * _L, _L)
                sv[e, pl.ds(j8 * _L, _L)] = a * (g[e, sl] + kv_r[e, sl])
            zero = jnp.zeros((_L,), jnp.float32)
            for j8 in range(DH):
                sv[_C + e, pl.ds(j8 * _L, _L)] = jnp.where(
                    s8 == jnp.float32(j8), a, zero)
            return ecarry

        lax.fori_loop(0, _C, _edge, 0)

    pltpu.sync_copy(sd_hbm.at[pl.ds(wid * _C, _C)], sd0)
    _build_gidx(0)
    _issue_gathers(0, wid)

    @pl.when(nj > 1)
    def _pro_idx1():
        pltpu.async_copy(sd_hbm.at[pl.ds((wid + _NW) * _C, _C)], sd1, sem_i)

    def _stage(j, p):
        q = 1 - p
        ci = j * _NW + wid
        _drain_gathers(p, ci)

        @pl.when(j >= 2)
        def _drain_scatter():
            pltpu.make_async_copy(sval[p], acc_sh.at[sidx[p]], sem_s).wait()

        _build_sidx(p)

        @pl.when(j + 1 < nj)
        def _prep_next():
            pltpu.make_async_copy(
                sd_hbm.at[pl.ds((ci + _NW) * _C, _C)], sd[q], sem_i).wait()
            _build_gidx(q)
            _issue_gathers(q, ci + _NW)

        @pl.when(j + 2 < nj)
        def _prefetch_idx():
            pltpu.async_copy(
                sd_hbm.at[pl.ds((ci + 2 * _NW) * _C, _C)], sd[p], sem_i)

        _compute(p)
        pltpu.async_copy(sval[p], acc_sh.at[sidx[p]], sem_s, add=True)

    def _pair(t, carry):
        for p in (0, 1):
            j = 2 * t + p

            @pl.when(j < nj)
            def _run():
                _stage(j, p)

        return carry

    lax.fori_loop(0, (nj + 1) // 2, _pair, 0)
    pltpu.make_async_copy(sval0, acc_sh.at[sidx0], sem_s).wait()
    pltpu.make_async_copy(sval1, acc_sh.at[sidx1], sem_s).wait()
    plsc.subcore_barrier()

    row0 = sid * _TROWS

    @pl.when(jnp.logical_not(last))
    def _out_main():
        pltpu.sync_copy(acc_sh.at[pl.ds(row0, _TROWS)],
                        num_out.at[cid, pl.ds(row0, _TROWS)])

    @pl.when(last)
    def _out_tail():
        pltpu.sync_copy(acc_sh.at[pl.ds(row0, _OUT_TAIL)],
                        num_out.at[cid, pl.ds(row0, _OUT_TAIL)])

    dbase = _NUMROWS + sid * 80
    ngrp = jnp.where(last, _OUT_TAIL // 64, _TROWS // 64)

    def _den_group(c, carry):
        pltpu.sync_copy(acc_sh.at[pl.ds(dbase + c * 8, 8)],
                        sval0.at[pl.ds(0, 8)])
        for qtr in range(4):
            def _unpack(i, carry2):
                n = qtr * _L + i
                denu[i, :] = sval0[n // DH, pl.ds((n % DH) * _L, _L)]
                return carry2

            lax.fori_loop(0, _L, _unpack, 0)
            pltpu.sync_copy(
                denu, den_out.at[cid, pl.ds(row0 + c * 64 + qtr * _L, _L)])
        return carry

    lax.fori_loop(0, ngrp, _den_group, 0)

    @pl.when(last)
    def _den_tail():
        pltpu.sync_copy(acc_sh.at[pl.ds(dbase + 48, 8)], sval0.at[pl.ds(0, 8)])

        def _unpack(i, carry2):
            denu[i, :] = sval0[i // DH, pl.ds((i % DH) * _L, _L)]
            return carry2

        lax.fori_loop(0, _L, _unpack, 0)
        pltpu.sync_copy(denu, den_out.at[cid, pl.ds(9984, _L)])


@functools.cache
def _build_edge_sc():
    return functools.partial(
        pl.kernel,
        out_type=(jax.ShapeDtypeStruct((_NC, N, D), jnp.float32),
                  jax.ShapeDtypeStruct((_NC, N, NH), jnp.float32)),
        mesh=plsc.VectorSubcoreMesh(core_axis_name="c", subcore_axis_name="s",
                                    num_cores=_NC, num_subcores=_NS),
        scratch_types=[
            pltpu.VMEM((_C,), jnp.int32),
            pltpu.VMEM((_C,), jnp.int32),
            pltpu.VMEM((2 * _C,), jnp.int32),
            pltpu.VMEM((2 * _C,), jnp.int32),
            pltpu.VMEM((2 * _C,), jnp.int32),
            pltpu.VMEM((2 * _C,), jnp.int32),
            pltpu.VMEM((2 * _C, 2 * D), jnp.float32),
            pltpu.VMEM((2 * _C, 2 * D), jnp.float32),
            pltpu.VMEM((_C, 2 * D), jnp.float32),
            pltpu.VMEM((_C, 2 * D), jnp.float32),
            pltpu.VMEM((2 * _C, D), jnp.float32),
            pltpu.VMEM((2 * _C, D), jnp.float32),
            pltpu.VMEM((_L, NH), jnp.float32),
            pltpu.VMEM_SHARED((_ACCROWS, D), jnp.float32),
            pltpu.SemaphoreType.DMA,
            pltpu.SemaphoreType.DMA,
            pltpu.SemaphoreType.DMA,
        ],
    )(_edge_sc_body)


_BN = 2000
_BE = 4000


def _rep_spec(shape):
    return pl.BlockSpec(shape, lambda i: (0,) * len(shape))


def kernel(x, r, edge_index, Wq, Wk, Wv, Wkr, Wvr, Wout, Wg, bg, Ws, bs,
           W1, b1, W2, b2, ln1_g, ln1_b, lnr_g, lnr_b, post_g, post_b,
           ffpre_g, ffpre_b, ffpost_g, ffpost_b):
    src = edge_index[0].astype(jnp.int32)
    dst = edge_index[1].astype(jnp.int32)

    def perm_cols(w):
        return w.reshape(D, NH, DH).transpose(0, 2, 1).reshape(D, NH * DH)

    wq_t = perm_cols(Wq)
    wk_t = perm_cols(Wk)
    wv_t = perm_cols(Wv)
    wkr_t = perm_cols(Wkr)
    wvr_t = perm_cols(Wvr)
    wout_p = Wout.reshape(NH, DH, D).transpose(1, 0, 2).reshape(NH * DH, D)
    wg1 = Wg[:D]
    wg2 = Wg[D:]

    def row(v):
        return v.reshape(1, -1)

    w_spec = _rep_spec((D, D))
    g_spec = _rep_spec((1, D))

    xn, pq, kv = pl.pallas_call(
        _pre_node_body,
        grid=(N // _BN,),
        in_specs=[pl.BlockSpec((_BN, D), lambda i: (i, 0)),
                  w_spec, w_spec, w_spec, g_spec, g_spec],
        out_specs=[pl.BlockSpec((_BN, D), lambda i: (i, 0)),
                   pl.BlockSpec((_BN, 2 * D), lambda i: (i, 0)),
                   pl.BlockSpec((_BN, 2 * D), lambda i: (i, 0))],
        out_shape=[jax.ShapeDtypeStruct((N, D), jnp.float32),
                   jax.ShapeDtypeStruct((N, 2 * D), jnp.float32),
                   jax.ShapeDtypeStruct((N, 2 * D), jnp.float32)],
    )(x, wq_t, wk_t, wv_t, row(ln1_g), row(ln1_b))

    kvr = pl.pallas_call(
        _pre_edge_body,
        grid=(E // _BE,),
        in_specs=[pl.BlockSpec((_BE, D), lambda i: (i, 0)),
                  w_spec, w_spec, g_spec, g_spec],
        out_specs=pl.BlockSpec((_BE, 2 * D), lambda i: (i, 0)),
        out_shape=jax.ShapeDtypeStruct((E, 2 * D), jnp.float32),
    )(r, wkr_t, wvr_t, row(lnr_g), row(lnr_b))

    kvpq = jnp.concatenate([kv, pq], axis=0)
    sd = jnp.bitwise_or(jnp.left_shift(src, 14), dst)
    num_p, den_p = _build_edge_sc()(sd, kvpq, kvr)

    y = pl.pallas_call(
        _post_body,
        grid=(N // _BN,),
        in_specs=[pl.BlockSpec((_BN, D), lambda i: (i, 0)),
                  pl.BlockSpec((_BN, D), lambda i: (i, 0)),
                  pl.BlockSpec((_NC, _BN, D), lambda i: (0, i, 0)),
                  pl.BlockSpec((_NC, _BN, NH), lambda i: (0, i, 0)),
                  w_spec, w_spec, w_spec, g_spec, w_spec, g_spec,
                  _rep_spec((D, F)), _rep_spec((1, F)),
                  _rep_spec((F, D)), g_spec,
                  g_spec, g_spec, g_spec, g_spec, g_spec, g_spec],
        out_specs=pl.BlockSpec((_BN, D), lambda i: (i, 0)),
        out_shape=jax.ShapeDtypeStruct((N, D), jnp.float32),
    )(x, xn, num_p, den_p, wout_p, wg1, wg2, row(bg), Ws, row(bs),
      W1, row(b1), W2, row(b2), row(post_g), row(post_b),
      row(ffpre_g), row(ffpre_b), row(ffpost_g), row(ffpost_b))

    return y


# speedup vs baseline: 82.5282x; 1.0284x over previous
import functools

import jax
import jax.numpy as jnp
from jax import lax
from jax.experimental import pallas as pl
from jax.experimental.pallas import tpu as pltpu
from jax.experimental.pallas import tpu_sc as plsc

N = 10000
E = 160000
D = 128
NH = 16
DH = 8
F = 4 * D

_EPS = 1e-5


def _layernorm(x, g, b):
    mu = jnp.mean(x, axis=-1, keepdims=True)
    var = jnp.mean((x - mu) ** 2, axis=-1, keepdims=True)
    return (x - mu) / jnp.sqrt(var + _EPS) * g + b


def _pre_node_body(x_ref, wq_ref, wk_ref, wv_ref, g_ref, b_ref,
                   xn_ref, pq_ref, k_ref, v_ref):
    xn = _layernorm(x_ref[...], g_ref[...], b_ref[...])
    xn_ref[...] = xn
    q = jnp.dot(xn, wq_ref[...], preferred_element_type=jnp.float32)
    pq = jnp.where(q > 0, q + 1.0, jnp.exp(q))
    nb = x_ref.shape[0]
    rows = lax.broadcasted_iota(jnp.int32, (nb, D), 0) + pl.program_id(0) * nb
    lg = lax.shift_right_logical(lax.broadcasted_iota(jnp.int32, (nb, D), 1), 4)
    sign = jnp.where(lg == jnp.bitwise_and(rows, DH - 1), -1.0, 1.0)
    pq_ref[...] = pq * sign
    k_ref[...] = jnp.dot(xn, wk_ref[...], preferred_element_type=jnp.float32)
    v_ref[...] = jnp.dot(xn, wv_ref[...], preferred_element_type=jnp.float32)


def _pre_edge_body(r_ref, wkr_ref, wvr_ref, g_ref, b_ref, kvr_ref):
    rn = _layernorm(r_ref[...], g_ref[...], b_ref[...])
    kr = jnp.dot(rn, wkr_ref[...], preferred_element_type=jnp.float32)
    vr = jnp.dot(rn, wvr_ref[...], preferred_element_type=jnp.float32)
    kvr_ref[...] = jnp.concatenate([kr, vr], axis=-1)


def _post_body(x_ref, xn_ref, num_ref, den_ref, wout_ref, wg1_ref, wg2_ref,
               bg_ref, ws_ref, bs_ref, w1_ref, b1_ref, w2_ref, b2_ref,
               postg_ref, postb_ref, ffpreg_ref, ffpreb_ref, ffpostg_ref,
               ffpostb_ref, y_ref):
    num = num_ref[0] + num_ref[1]
    den = jnp.maximum(den_ref[0] + den_ref[1], 1e-6)
    den_full = jnp.concatenate([den] * DH, axis=-1)
    attn = num / den_full
    out = jnp.dot(attn, wout_ref[...], preferred_element_type=jnp.float32)
    xn = xn_ref[...]
    gate_pre = (jnp.dot(out, wg1_ref[...], preferred_element_type=jnp.float32)
                + jnp.dot(xn, wg2_ref[...], preferred_element_type=jnp.float32)
                + bg_ref[...])
    g = jax.nn.sigmoid(gate_pre)
    skip = jnp.dot(xn, ws_ref[...], preferred_element_type=jnp.float32) + bs_ref[...]
    out = out + g * (skip - out)
    x_mid = x_ref[...] + _layernorm(out, postg_ref[...], postb_ref[...])
    h = _layernorm(x_mid, ffpreg_ref[...], ffpreb_ref[...])
    h = jnp.dot(h, w1_ref[...], preferred_element_type=jnp.float32) + b1_ref[...]
    h = 0.5 * h * (1.0 + lax.erf(h * (2.0 ** -0.5)))
    h = jnp.dot(h, w2_ref[...], preferred_element_type=jnp.float32) + b2_ref[...]
    y_ref[...] = x_mid + _layernorm(h, ffpostg_ref[...], ffpostb_ref[...])


_C = 16
_NCHUNK = E // _C
_NC = 2
_NS = 16
_NW = _NC * _NS
_L = 16
_NUMROWS = 10240
_DROWS = 1280
_ACCROWS = _NUMROWS + _DROWS
_TROWS = 640
_OUT_TAIL = N - 15 * _TROWS
_ZROWS = _ACCROWS // _NS


def _edge_sc_body(sd_hbm, kvpq_hbm, kvr_hbm, num_out, den_out,
                  sd0, sd1, gidx0, gidx1, sidx0, sidx1,
                  grow0, grow1, kvr0, kvr1, sval0, sval1, denu,
                  acc_sh, sem_i, sem_g, sem_s):
    cid = lax.axis_index("c")
    sid = lax.axis_index("s")
    wid = sid * _NC + cid
    last = sid == _NS - 1
    sd = (sd0, sd1)
    gidx = (gidx0, gidx1)
    sidx = (sidx0, sidx1)
    grow = (grow0, grow1)
    kvrb = (kvr0, kvr1)
    sval = (sval0, sval1)
    nj = (_NCHUNK - wid + _NW - 1) // _NW

    def _zero_row(i, carry):
        for j in range(DH):
            sval0[i, pl.ds(j * _L, _L)] = jnp.zeros((_L,), jnp.float32)
        return carry

    lax.fori_loop(0, 2 * _C, _zero_row, 0)
    z0 = sid * _ZROWS

    def _zero_acc(i, carry):
        pltpu.sync_copy(sval0, acc_sh.at[pl.ds(z0 + i * 32, 32)])
        return carry

    lax.fori_loop(0, _ZROWS // 32, _zero_acc, 0)
    pltpu.sync_copy(sval0.at[pl.ds(0, _ZROWS % 32)],
                    acc_sh.at[pl.ds(z0 + (_ZROWS // 32) * 32, _ZROWS % 32)])
    plsc.subcore_barrier()

    def _build_gidx(p):
        sdv = sd[p][...]
        srcv = lax.shift_right_logical(sdv, 14)
        gidx[p][pl.ds(0, _L)] = srcv
        gidx[p][pl.ds(_L, _L)] = N + srcv
        gidx[p][pl.ds(2 * _L, _L)] = 2 * N + jnp.bitwise_and(sdv, 16383)

    def _build_sidx(p):
        sdv = sd[p][...]
        dstv = jnp.bitwise_and(sdv, 16383)
        sidx[p][pl.ds(0, _L)] = dstv
        sidx[p][pl.ds(_L, _L)] = _NUMROWS + lax.shift_right_logical(dstv, 3)

    def _issue_gathers(p, ci):
        pltpu.async_copy(kvpq_hbm.at[gidx[p]], grow[p], sem_g)
        pltpu.async_copy(kvr_hbm.at[pl.ds(ci * _C, _C)], kvrb[p], sem_g)

    def _drain_gathers(p, ci):
        pltpu.make_async_copy(kvpq_hbm.at[gidx[p]], grow[p], sem_g).wait()
        pltpu.make_async_copy(kvr_hbm.at[pl.ds(ci * _C, _C)], kvrb[p],
                              sem_g).wait()

    def _compute(p):
        g = grow[p]
        kv_r = kvrb[p]
        sv = sval[p]

        def _edge(e, ecarry):
            a = jnp.zeros((_L,), jnp.float32)
            qs = []
            for j8 in range(DH):
                sl = pl.ds(j8 * _L, _L)
                ke = g[e, sl] + kv_r[e, sl]
                pk = jnp.where(ke > 0, ke + 1.0, jnp.exp(ke))
                q16 = g[2 * _C + e, sl]
                qs.append(q16)
                a = a + jnp.abs(q16) * pk
            for j8 in range(DH):
                sv[e, pl.ds(j8 * _L, _L)] = a * (
                    g[_C + e, pl.ds(j8 * _L, _L)]
                    + kv_r[e, pl.ds(D + j8 * _L, _L)])
            zero = jnp.zeros((_L,), jnp.float32)
            for j8 in range(DH):
                sv[_C + e, pl.ds(j8 * _L, _L)] = jnp.where(
                    qs[j8] < 0, a, zero)
            return ecarry

        lax.fori_loop(0, _C, _edge, 0)

    pltpu.sync_copy(sd_hbm.at[pl.ds(wid * _C, _C)], sd0)
    _build_gidx(0)
    _issue_gathers(0, wid)

    @pl.when(nj > 1)
    def _pro_idx1():
        pltpu.async_copy(sd_hbm.at[pl.ds((wid + _NW) * _C, _C)], sd1, sem_i)

    def _stage(j, p):
        q = 1 - p
        ci = j * _NW + wid
        _drain_gathers(p, ci)

        @pl.when(j >= 2)
        def _drain_scatter():
            pltpu.make_async_copy(sval[p], acc_sh.at[sidx[p]], sem_s).wait()

        _build_sidx(p)

        @pl.when(j + 1 < nj)
        def _prep_next():
            pltpu.make_async_copy(
                sd_hbm.at[pl.ds((ci + _NW) * _C, _C)], sd[q], sem_i).wait()
            _build_gidx(q)
            _issue_gathers(q, ci + _NW)

        @pl.when(j + 2 < nj)
        def _prefetch_idx():
            pltpu.async_copy(
                sd_hbm.at[pl.ds((ci + 2 * _NW) * _C, _C)], sd[p], sem_i)

        _compute(p)
        pltpu.async_copy(sval[p], acc_sh.at[sidx[p]], sem_s, add=True)

    def _pair(t, carry):
        for p in (0, 1):
            j = 2 * t + p

            @pl.when(j < nj)
            def _run():
                _stage(j, p)

        return carry

    lax.fori_loop(0, (nj + 1) // 2, _pair, 0)
    pltpu.make_async_copy(sval0, acc_sh.at[sidx0], sem_s).wait()
    pltpu.make_async_copy(sval1, acc_sh.at[sidx1], sem_s).wait()
    plsc.subcore_barrier()

    row0 = sid * _TROWS

    @pl.when(jnp.logical_not(last))
    def _out_main():
        pltpu.sync_copy(acc_sh.at[pl.ds(row0, _TROWS)],
                        num_out.at[cid, pl.ds(row0, _TROWS)])

    @pl.when(last)
    def _out_tail():
        pltpu.sync_copy(acc_sh.at[pl.ds(row0, _OUT_TAIL)],
                        num_out.at[cid, pl.ds(row0, _OUT_TAIL)])

    dbase = _NUMROWS + sid * 80
    ngrp = jnp.where(last, _OUT_TAIL // 64, _TROWS // 64)

    def _den_group(c, carry):
        pltpu.sync_copy(acc_sh.at[pl.ds(dbase + c * 8, 8)],
                        sval0.at[pl.ds(0, 8)])
        for qtr in range(4):
            def _unpack(i, carry2):
                n = qtr * _L + i
                denu[i, :] = sval0[n // DH, pl.ds((n % DH) * _L, _L)]
                return carry2

            lax.fori_loop(0, _L, _unpack, 0)
            pltpu.sync_copy(
                denu, den_out.at[cid, pl.ds(row0 + c * 64 + qtr * _L, _L)])
        return carry

    lax.fori_loop(0, ngrp, _den_group, 0)

    @pl.when(last)
    def _den_tail():
        pltpu.sync_copy(acc_sh.at[pl.ds(dbase + 48, 8)], sval0.at[pl.ds(0, 8)])

        def _unpack(i, carry2):
            denu[i, :] = sval0[i // DH, pl.ds((i % DH) * _L, _L)]
            return carry2

        lax.fori_loop(0, _L, _unpack, 0)
        pltpu.sync_copy(denu, den_out.at[cid, pl.ds(9984, _L)])


@functools.cache
def _build_edge_sc():
    return functools.partial(
        pl.kernel,
        out_type=(jax.ShapeDtypeStruct((_NC, N, D), jnp.float32),
                  jax.ShapeDtypeStruct((_NC, N, NH), jnp.float32)),
        mesh=plsc.VectorSubcoreMesh(core_axis_name="c", subcore_axis_name="s",
                                    num_cores=_NC, num_subcores=_NS),
        scratch_types=[
            pltpu.VMEM((_C,), jnp.int32),
            pltpu.VMEM((_C,), jnp.int32),
            pltpu.VMEM((3 * _C,), jnp.int32),
            pltpu.VMEM((3 * _C,), jnp.int32),
            pltpu.VMEM((2 * _C,), jnp.int32),
            pltpu.VMEM((2 * _C,), jnp.int32),
            pltpu.VMEM((3 * _C, D), jnp.float32),
            pltpu.VMEM((3 * _C, D), jnp.float32),
            pltpu.VMEM((_C, 2 * D), jnp.float32),
            pltpu.VMEM((_C, 2 * D), jnp.float32),
            pltpu.VMEM((2 * _C, D), jnp.float32),
            pltpu.VMEM((2 * _C, D), jnp.float32),
            pltpu.VMEM((_L, NH), jnp.float32),
            pltpu.VMEM_SHARED((_ACCROWS, D), jnp.float32),
            pltpu.SemaphoreType.DMA,
            pltpu.SemaphoreType.DMA,
            pltpu.SemaphoreType.DMA,
        ],
    )(_edge_sc_body)


_BN = 2000
_BE = 4000


def _rep_spec(shape):
    return pl.BlockSpec(shape, lambda i: (0,) * len(shape))


def kernel(x, r, edge_index, Wq, Wk, Wv, Wkr, Wvr, Wout, Wg, bg, Ws, bs,
           W1, b1, W2, b2, ln1_g, ln1_b, lnr_g, lnr_b, post_g, post_b,
           ffpre_g, ffpre_b, ffpost_g, ffpost_b):
    src = edge_index[0].astype(jnp.int32)
    dst = edge_index[1].astype(jnp.int32)

    def perm_cols(w):
        return w.reshape(D, NH, DH).transpose(0, 2, 1).reshape(D, NH * DH)

    wq_t = perm_cols(Wq)
    wk_t = perm_cols(Wk)
    wv_t = perm_cols(Wv)
    wkr_t = perm_cols(Wkr)
    wvr_t = perm_cols(Wvr)
    wout_p = Wout.reshape(NH, DH, D).transpose(1, 0, 2).reshape(NH * DH, D)
    wg1 = Wg[:D]
    wg2 = Wg[D:]

    def row(v):
        return v.reshape(1, -1)

    w_spec = _rep_spec((D, D))
    g_spec = _rep_spec((1, D))

    xn, pq, k, v = pl.pallas_call(
        _pre_node_body,
        grid=(N // _BN,),
        in_specs=[pl.BlockSpec((_BN, D), lambda i: (i, 0)),
                  w_spec, w_spec, w_spec, g_spec, g_spec],
        out_specs=[pl.BlockSpec((_BN, D), lambda i: (i, 0))] * 4,
        out_shape=[jax.ShapeDtypeStruct((N, D), jnp.float32)] * 4,
    )(x, wq_t, wk_t, wv_t, row(ln1_g), row(ln1_b))

    kvr = pl.pallas_call(
        _pre_edge_body,
        grid=(E // _BE,),
        in_specs=[pl.BlockSpec((_BE, D), lambda i: (i, 0)),
                  w_spec, w_spec, g_spec, g_spec],
        out_specs=pl.BlockSpec((_BE, 2 * D), lambda i: (i, 0)),
        out_shape=jax.ShapeDtypeStruct((E, 2 * D), jnp.float32),
    )(r, wkr_t, wvr_t, row(lnr_g), row(lnr_b))

    kvpq = jnp.concatenate([k, v, pq], axis=0)
    sd = jnp.bitwise_or(jnp.left_shift(src, 14), dst)
    num_p, den_p = _build_edge_sc()(sd, kvpq, kvr)

    y = pl.pallas_call(
        _post_body,
        grid=(N // _BN,),
        in_specs=[pl.BlockSpec((_BN, D), lambda i: (i, 0)),
                  pl.BlockSpec((_BN, D), lambda i: (i, 0)),
                  pl.BlockSpec((_NC, _BN, D), lambda i: (0, i, 0)),
                  pl.BlockSpec((_NC, _BN, NH), lambda i: (0, i, 0)),
                  w_spec, w_spec, w_spec, g_spec, w_spec, g_spec,
                  _rep_spec((D, F)), _rep_spec((1, F)),
                  _rep_spec((F, D)), g_spec,
                  g_spec, g_spec, g_spec, g_spec, g_spec, g_spec],
        out_specs=pl.BlockSpec((_BN, D), lambda i: (i, 0)),
        out_shape=jax.ShapeDtypeStruct((N, D), jnp.float32),
    )(x, xn, num_p, den_p, wout_p, wg1, wg2, row(bg), Ws, row(bs),
      W1, row(b1), W2, row(b2), row(post_g), row(post_b),
      row(ffpre_g), row(ffpre_b), row(ffpost_g), row(ffpost_b))

    return y


# speedup vs baseline: 84.1640x; 1.0198x over previous
import functools

import jax
import jax.numpy as jnp
from jax import lax
from jax.experimental import pallas as pl
from jax.experimental.pallas import tpu as pltpu
from jax.experimental.pallas import tpu_sc as plsc

N = 10000
E = 160000
D = 128
NH = 16
DH = 8
F = 4 * D

_EPS = 1e-5


def _layernorm(x, g, b):
    mu = jnp.mean(x, axis=-1, keepdims=True)
    var = jnp.mean((x - mu) ** 2, axis=-1, keepdims=True)
    return (x - mu) / jnp.sqrt(var + _EPS) * g + b


def _pre_node_body(x_ref, wq_ref, wk_ref, wv_ref, g_ref, b_ref,
                   xn_ref, pq_ref, k_ref, v_ref):
    xn = _layernorm(x_ref[...], g_ref[...], b_ref[...])
    xn_ref[...] = xn
    q = jnp.dot(xn, wq_ref[...], preferred_element_type=jnp.float32)
    pq = jnp.where(q > 0, q + 1.0, jnp.exp(q))
    nb = x_ref.shape[0]
    rows = lax.broadcasted_iota(jnp.int32, (nb, D), 0) + pl.program_id(0) * nb
    lg = lax.shift_right_logical(lax.broadcasted_iota(jnp.int32, (nb, D), 1), 4)
    sign = jnp.where(lg == jnp.bitwise_and(rows, DH - 1), -1.0, 1.0)
    pq_ref[...] = pq * sign
    k_ref[...] = jnp.dot(xn, wk_ref[...], preferred_element_type=jnp.float32)
    v_ref[...] = jnp.dot(xn, wv_ref[...], preferred_element_type=jnp.float32)


def _pre_edge_body(r_ref, wkr_ref, wvr_ref, g_ref, b_ref, kvr_ref):
    rn = _layernorm(r_ref[...], g_ref[...], b_ref[...])
    kr = jnp.dot(rn, wkr_ref[...], preferred_element_type=jnp.float32)
    vr = jnp.dot(rn, wvr_ref[...], preferred_element_type=jnp.float32)
    kvr_ref[...] = jnp.concatenate([kr, vr], axis=-1)


def _post_body(x_ref, xn_ref, num_ref, den_ref, wout_ref, wg1_ref, wg2_ref,
               bg_ref, ws_ref, bs_ref, w1_ref, b1_ref, w2_ref, b2_ref,
               postg_ref, postb_ref, ffpreg_ref, ffpreb_ref, ffpostg_ref,
               ffpostb_ref, y_ref):
    num = num_ref[0] + num_ref[1]
    den = jnp.maximum(den_ref[0] + den_ref[1], 1e-6)
    den_full = jnp.concatenate([den] * DH, axis=-1)
    attn = num / den_full
    out = jnp.dot(attn, wout_ref[...], preferred_element_type=jnp.float32)
    xn = xn_ref[...]
    gate_pre = (jnp.dot(out, wg1_ref[...], preferred_element_type=jnp.float32)
                + jnp.dot(xn, wg2_ref[...], preferred_element_type=jnp.float32)
                + bg_ref[...])
    g = jax.nn.sigmoid(gate_pre)
    skip = jnp.dot(xn, ws_ref[...], preferred_element_type=jnp.float32) + bs_ref[...]
    out = out + g * (skip - out)
    x_mid = x_ref[...] + _layernorm(out, postg_ref[...], postb_ref[...])
    h = _layernorm(x_mid, ffpreg_ref[...], ffpreb_ref[...])
    h = jnp.dot(h, w1_ref[...], preferred_element_type=jnp.float32) + b1_ref[...]
    h = 0.5 * h * (1.0 + lax.erf(h * (2.0 ** -0.5)))
    h = jnp.dot(h, w2_ref[...], preferred_element_type=jnp.float32) + b2_ref[...]
    y_ref[...] = x_mid + _layernorm(h, ffpostg_ref[...], ffpostb_ref[...])


_C = 16
_NCHUNK = E // _C
_NC = 2
_NS = 16
_NW = _NC * _NS
_L = 16
_NUMROWS = 10240
_DROWS = 1280
_ACCROWS = _NUMROWS + _DROWS
_TROWS = 640
_OUT_TAIL = N - 15 * _TROWS
_ZROWS = _ACCROWS // _NS


def _edge_sc_body(sd_hbm, kvpq_hbm, kvr_hbm, num_out, den_out,
                  sd0, sd1, gidx0, gidx1, sidx0, sidx1,
                  grow0, grow1, kvr0, kvr1, sval0, sval1, denu,
                  acc_sh, sem_i, sem_g, sem_s):
    cid = lax.axis_index("c")
    sid = lax.axis_index("s")
    wid = sid * _NC + cid
    last = sid == _NS - 1
    sd = (sd0, sd1)
    gidx = (gidx0, gidx1)
    sidx = (sidx0, sidx1)
    grow = (grow0, grow1)
    kvrb = (kvr0, kvr1)
    sval = (sval0, sval1)
    nj = (_NCHUNK - wid + _NW - 1) // _NW

    def _zero_row(i, carry):
        for j in range(DH):
            sval0[i, pl.ds(j * _L, _L)] = jnp.zeros((_L,), jnp.float32)
        return carry

    lax.fori_loop(0, 2 * _C, _zero_row, 0)
    z0 = sid * _ZROWS

    def _zero_acc(i, carry):
        pltpu.sync_copy(sval0, acc_sh.at[pl.ds(z0 + i * 32, 32)])
        return carry

    lax.fori_loop(0, _ZROWS // 32, _zero_acc, 0)
    pltpu.sync_copy(sval0.at[pl.ds(0, _ZROWS % 32)],
                    acc_sh.at[pl.ds(z0 + (_ZROWS // 32) * 32, _ZROWS % 32)])
    plsc.subcore_barrier()

    def _build_gidx(p):
        sdv = sd[p][...]
        srcv = lax.shift_right_logical(sdv, 14)
        gidx[p][pl.ds(0, _L)] = srcv
        gidx[p][pl.ds(_L, _L)] = N + srcv
        gidx[p][pl.ds(2 * _L, _L)] = 2 * N + jnp.bitwise_and(sdv, 16383)

    def _build_sidx(p):
        sdv = sd[p][...]
        dstv = jnp.bitwise_and(sdv, 16383)
        sidx[p][pl.ds(0, _L)] = dstv
        sidx[p][pl.ds(_L, _L)] = _NUMROWS + lax.shift_right_logical(dstv, 3)

    def _issue_gathers(p, ci):
        pltpu.async_copy(kvpq_hbm.at[gidx[p]], grow[p], sem_g)
        pltpu.async_copy(kvr_hbm.at[pl.ds(ci * _C, _C)], kvrb[p], sem_g)

    def _drain_gathers(p, ci):
        pltpu.make_async_copy(kvpq_hbm.at[gidx[p]], grow[p], sem_g).wait()
        pltpu.make_async_copy(kvr_hbm.at[pl.ds(ci * _C, _C)], kvrb[p],
                              sem_g).wait()

    def _compute(p):
        g = grow[p]
        kv_r = kvrb[p]
        sv = sval[p]

        def _edge(e, ecarry):
            a = jnp.zeros((_L,), jnp.float32)
            qs = []
            for j8 in range(DH):
                sl = pl.ds(j8 * _L, _L)
                ke = g[e, sl] + kv_r[e, sl]
                pk = jnp.where(ke > 0, ke + 1.0, jnp.exp(ke))
                q16 = g[2 * _C + e, sl]
                qs.append(q16)
                a = a + jnp.abs(q16) * pk
            for j8 in range(DH):
                sv[e, pl.ds(j8 * _L, _L)] = a * (
                    g[_C + e, pl.ds(j8 * _L, _L)]
                    + kv_r[e, pl.ds(D + j8 * _L, _L)])
            zero = jnp.zeros((_L,), jnp.float32)
            for j8 in range(DH):
                sv[_C + e, pl.ds(j8 * _L, _L)] = jnp.where(
                    qs[j8] < 0, a, zero)
            return ecarry

        lax.fori_loop(0, _C, _edge, 0)

    pltpu.sync_copy(sd_hbm.at[pl.ds(wid * _C, _C)], sd0)
    _build_gidx(0)
    _issue_gathers(0, wid)

    @pl.when(nj > 1)
    def _pro_idx1():
        pltpu.async_copy(sd_hbm.at[pl.ds((wid + _NW) * _C, _C)], sd1, sem_i)

    def _stage(j, p):
        q = 1 - p
        ci = j * _NW + wid
        _drain_gathers(p, ci)

        @pl.when(j >= 2)
        def _drain_scatter():
            pltpu.make_async_copy(sval[p], acc_sh.at[sidx[p]], sem_s).wait()

        _build_sidx(p)

        @pl.when(j + 1 < nj)
        def _prep_next():
            pltpu.make_async_copy(
                sd_hbm.at[pl.ds((ci + _NW) * _C, _C)], sd[q], sem_i).wait()
            _build_gidx(q)
            _issue_gathers(q, ci + _NW)

        @pl.when(j + 2 < nj)
        def _prefetch_idx():
            pltpu.async_copy(
                sd_hbm.at[pl.ds((ci + 2 * _NW) * _C, _C)], sd[p], sem_i)

        _compute(p)
        pltpu.async_copy(sval[p], acc_sh.at[sidx[p]], sem_s, add=True)

    def _pair(t, carry):
        for p in (0, 1):
            j = 2 * t + p

            @pl.when(j < nj)
            def _run():
                _stage(j, p)

        return carry

    lax.fori_loop(0, (nj + 1) // 2, _pair, 0)
    pltpu.make_async_copy(sval0, acc_sh.at[sidx0], sem_s).wait()
    pltpu.make_async_copy(sval1, acc_sh.at[sidx1], sem_s).wait()
    plsc.subcore_barrier()

    row0 = sid * _TROWS

    @pl.when(jnp.logical_not(last))
    def _out_main():
        pltpu.sync_copy(acc_sh.at[pl.ds(row0, _TROWS)],
                        num_out.at[cid, pl.ds(row0, _TROWS)])

    @pl.when(last)
    def _out_tail():
        pltpu.sync_copy(acc_sh.at[pl.ds(row0, _OUT_TAIL)],
                        num_out.at[cid, pl.ds(row0, _OUT_TAIL)])

    dbase = _NUMROWS + sid * 80
    ngrp = jnp.where(last, _OUT_TAIL // 64, _TROWS // 64)

    def _den_group(c, carry):
        pltpu.sync_copy(acc_sh.at[pl.ds(dbase + c * 8, 8)],
                        sval0.at[pl.ds(0, 8)])
        for qtr in range(4):
            def _unpack(i, carry2):
                n = qtr * _L + i
                denu[i, :] = sval0[n // DH, pl.ds((n % DH) * _L, _L)]
                return carry2

            lax.fori_loop(0, _L, _unpack, 0)
            pltpu.sync_copy(
                denu, den_out.at[cid, pl.ds(row0 + c * 64 + qtr * _L, _L)])
        return carry

    lax.fori_loop(0, ngrp, _den_group, 0)

    @pl.when(last)
    def _den_tail():
        pltpu.sync_copy(acc_sh.at[pl.ds(dbase + 48, 8)], sval0.at[pl.ds(0, 8)])

        def _unpack(i, carry2):
            denu[i, :] = sval0[i // DH, pl.ds((i % DH) * _L, _L)]
            return carry2

        lax.fori_loop(0, _L, _unpack, 0)
        pltpu.sync_copy(denu, den_out.at[cid, pl.ds(9984, _L)])


@functools.cache
def _build_edge_sc():
    return functools.partial(
        pl.kernel,
        out_type=(jax.ShapeDtypeStruct((_NC, N, D), jnp.float32),
                  jax.ShapeDtypeStruct((_NC, N, NH), jnp.float32)),
        mesh=plsc.VectorSubcoreMesh(core_axis_name="c", subcore_axis_name="s",
                                    num_cores=_NC, num_subcores=_NS),
        scratch_types=[
            pltpu.VMEM((_C,), jnp.int32),
            pltpu.VMEM((_C,), jnp.int32),
            pltpu.VMEM((3 * _C,), jnp.int32),
            pltpu.VMEM((3 * _C,), jnp.int32),
            pltpu.VMEM((2 * _C,), jnp.int32),
            pltpu.VMEM((2 * _C,), jnp.int32),
            pltpu.VMEM((3 * _C, D), jnp.float32),
            pltpu.VMEM((3 * _C, D), jnp.float32),
            pltpu.VMEM((_C, 2 * D), jnp.float32),
            pltpu.VMEM((_C, 2 * D), jnp.float32),
            pltpu.VMEM((2 * _C, D), jnp.float32),
            pltpu.VMEM((2 * _C, D), jnp.float32),
            pltpu.VMEM((_L, NH), jnp.float32),
            pltpu.VMEM_SHARED((_ACCROWS, D), jnp.float32),
            pltpu.SemaphoreType.DMA,
            pltpu.SemaphoreType.DMA,
            pltpu.SemaphoreType.DMA,
        ],
    )(_edge_sc_body)


_BN = 2000
_BE = 8000


def _rep_spec(shape):
    return pl.BlockSpec(shape, lambda i: (0,) * len(shape))


def kernel(x, r, edge_index, Wq, Wk, Wv, Wkr, Wvr, Wout, Wg, bg, Ws, bs,
           W1, b1, W2, b2, ln1_g, ln1_b, lnr_g, lnr_b, post_g, post_b,
           ffpre_g, ffpre_b, ffpost_g, ffpost_b):
    src = edge_index[0].astype(jnp.int32)
    dst = edge_index[1].astype(jnp.int32)

    def perm_cols(w):
        return w.reshape(D, NH, DH).transpose(0, 2, 1).reshape(D, NH * DH)

    wq_t = perm_cols(Wq)
    wk_t = perm_cols(Wk)
    wv_t = perm_cols(Wv)
    wkr_t = perm_cols(Wkr)
    wvr_t = perm_cols(Wvr)
    wout_p = Wout.reshape(NH, DH, D).transpose(1, 0, 2).reshape(NH * DH, D)
    wg1 = Wg[:D]
    wg2 = Wg[D:]

    def row(v):
        return v.reshape(1, -1)

    w_spec = _rep_spec((D, D))
    g_spec = _rep_spec((1, D))

    xn, pq, k, v = pl.pallas_call(
        _pre_node_body,
        grid=(N // _BN,),
        in_specs=[pl.BlockSpec((_BN, D), lambda i: (i, 0)),
                  w_spec, w_spec, w_spec, g_spec, g_spec],
        out_specs=[pl.BlockSpec((_BN, D), lambda i: (i, 0))] * 4,
        out_shape=[jax.ShapeDtypeStruct((N, D), jnp.float32)] * 4,
    )(x, wq_t, wk_t, wv_t, row(ln1_g), row(ln1_b))

    kvr = pl.pallas_call(
        _pre_edge_body,
        grid=(E // _BE,),
        in_specs=[pl.BlockSpec((_BE, D), lambda i: (i, 0)),
                  w_spec, w_spec, g_spec, g_spec],
        out_specs=pl.BlockSpec((_BE, 2 * D), lambda i: (i, 0)),
        out_shape=jax.ShapeDtypeStruct((E, 2 * D), jnp.float32),
    )(r, wkr_t, wvr_t, row(lnr_g), row(lnr_b))

    kvpq = jnp.concatenate([k, v, pq], axis=0)
    sd = jnp.bitwise_or(jnp.left_shift(src, 14), dst)
    num_p, den_p = _build_edge_sc()(sd, kvpq, kvr)

    y = pl.pallas_call(
        _post_body,
        grid=(N // _BN,),
        in_specs=[pl.BlockSpec((_BN, D), lambda i: (i, 0)),
                  pl.BlockSpec((_BN, D), lambda i: (i, 0)),
                  pl.BlockSpec((_NC, _BN, D), lambda i: (0, i, 0)),
                  pl.BlockSpec((_NC, _BN, NH), lambda i: (0, i, 0)),
                  w_spec, w_spec, w_spec, g_spec, w_spec, g_spec,
                  _rep_spec((D, F)), _rep_spec((1, F)),
                  _rep_spec((F, D)), g_spec,
                  g_spec, g_spec, g_spec, g_spec, g_spec, g_spec],
        out_specs=pl.BlockSpec((_BN, D), lambda i: (i, 0)),
        out_shape=jax.ShapeDtypeStruct((N, D), jnp.float32),
    )(x, xn, num_p, den_p, wout_p, wg1, wg2, row(bg), Ws, row(bs),
      W1, row(b1), W2, row(b2), row(post_g), row(post_b),
      row(ffpre_g), row(ffpre_b), row(ffpost_g), row(ffpost_b))

    return y


# speedup vs baseline: 87.0890x; 1.0348x over previous
import functools

import jax
import jax.numpy as jnp
from jax import lax
from jax.experimental import pallas as pl
from jax.experimental.pallas import tpu as pltpu
from jax.experimental.pallas import tpu_sc as plsc

N = 10000
E = 160000
D = 128
NH = 16
DH = 8
F = 4 * D

_EPS = 1e-5


def _layernorm(x, g, b):
    mu = jnp.mean(x, axis=-1, keepdims=True)
    var = jnp.mean((x - mu) ** 2, axis=-1, keepdims=True)
    return (x - mu) / jnp.sqrt(var + _EPS) * g + b


def _pre_node_body(x_ref, wq_ref, wk_ref, wv_ref, g_ref, b_ref,
                   xn_ref, pq_ref, k_ref, v_ref):
    xn = _layernorm(x_ref[...], g_ref[...], b_ref[...])
    xn_ref[...] = xn
    q = jnp.dot(xn, wq_ref[...], preferred_element_type=jnp.float32)
    pq = jnp.where(q > 0, q + 1.0, jnp.exp(q))
    nb = x_ref.shape[0]
    rows = lax.broadcasted_iota(jnp.int32, (nb, D), 0) + pl.program_id(0) * nb
    lg = lax.shift_right_logical(lax.broadcasted_iota(jnp.int32, (nb, D), 1), 4)
    sign = jnp.where(lg == jnp.bitwise_and(rows, DH - 1), -1.0, 1.0)
    pq_ref[...] = pq * sign
    k_ref[...] = jnp.dot(xn, wk_ref[...], preferred_element_type=jnp.float32)
    v_ref[...] = jnp.dot(xn, wv_ref[...], preferred_element_type=jnp.float32)


def _pre_edge_body(r_ref, wkr_ref, wvr_ref, g_ref, b_ref, kvr_ref):
    rn = _layernorm(r_ref[...], g_ref[...], b_ref[...])
    kr = jnp.dot(rn, wkr_ref[...], preferred_element_type=jnp.float32)
    vr = jnp.dot(rn, wvr_ref[...], preferred_element_type=jnp.float32)
    kvr_ref[...] = jnp.concatenate([kr, vr], axis=-1)


def _post_body(x_ref, xn_ref, num_ref, den_ref, wout_ref, wg1_ref, wg2_ref,
               bg_ref, ws_ref, bs_ref, w1_ref, b1_ref, w2_ref, b2_ref,
               postg_ref, postb_ref, ffpreg_ref, ffpreb_ref, ffpostg_ref,
               ffpostb_ref, y_ref):
    num = num_ref[0] + num_ref[1]
    den = jnp.maximum(den_ref[0] + den_ref[1], 1e-6)
    den_full = jnp.concatenate([den] * DH, axis=-1)
    attn = num / den_full
    out = jnp.dot(attn, wout_ref[...], preferred_element_type=jnp.float32)
    xn = xn_ref[...]
    gate_pre = (jnp.dot(out, wg1_ref[...], preferred_element_type=jnp.float32)
                + jnp.dot(xn, wg2_ref[...], preferred_element_type=jnp.float32)
                + bg_ref[...])
    g = jax.nn.sigmoid(gate_pre)
    skip = jnp.dot(xn, ws_ref[...], preferred_element_type=jnp.float32) + bs_ref[...]
    out = out + g * (skip - out)
    x_mid = x_ref[...] + _layernorm(out, postg_ref[...], postb_ref[...])
    h = _layernorm(x_mid, ffpreg_ref[...], ffpreb_ref[...])
    h = jnp.dot(h, w1_ref[...], preferred_element_type=jnp.float32) + b1_ref[...]
    h = 0.5 * h * (1.0 + lax.erf(h * (2.0 ** -0.5)))
    h = jnp.dot(h, w2_ref[...], preferred_element_type=jnp.float32) + b2_ref[...]
    y_ref[...] = x_mid + _layernorm(h, ffpostg_ref[...], ffpostb_ref[...])


_C = 16
_NCHUNK = E // _C
_NC = 2
_NS = 16
_NW = _NC * _NS
_L = 16
_NUMROWS = 10240
_DROWS = 1280
_ACCROWS = _NUMROWS + _DROWS
_TROWS = 640
_OUT_TAIL = N - 15 * _TROWS
_ZROWS = _ACCROWS // _NS


def _edge_sc_body(sd_hbm, kvpq_hbm, kvr_hbm, num_out, den_out,
                  sd0, sd1, gidx0, gidx1, sidx0, sidx1,
                  grow0, grow1, kvr0, kvr1, sval0, sval1, denu,
                  acc_sh, sem_i, sem_g, sem_s):
    cid = lax.axis_index("c")
    sid = lax.axis_index("s")
    wid = sid * _NC + cid
    last = sid == _NS - 1
    sd = (sd0, sd1)
    gidx = (gidx0, gidx1)
    sidx = (sidx0, sidx1)
    grow = (grow0, grow1)
    kvrb = (kvr0, kvr1)
    sval = (sval0, sval1)
    nj = (_NCHUNK - wid + _NW - 1) // _NW

    def _zero_row(i, carry):
        for j in range(DH):
            sval0[i, pl.ds(j * _L, _L)] = jnp.zeros((_L,), jnp.float32)
        return carry

    lax.fori_loop(0, 2 * _C, _zero_row, 0)
    z0 = sid * _ZROWS

    def _zero_acc(i, carry):
        pltpu.sync_copy(sval0, acc_sh.at[pl.ds(z0 + i * 32, 32)])
        return carry

    lax.fori_loop(0, _ZROWS // 32, _zero_acc, 0)
    pltpu.sync_copy(sval0.at[pl.ds(0, _ZROWS % 32)],
                    acc_sh.at[pl.ds(z0 + (_ZROWS // 32) * 32, _ZROWS % 32)])
    plsc.subcore_barrier()

    def _build_gidx(p):
        sdv = sd[p][...]
        srcv = lax.shift_right_logical(sdv, 14)
        gidx[p][pl.ds(0, _L)] = srcv
        gidx[p][pl.ds(_L, _L)] = N + srcv
        gidx[p][pl.ds(2 * _L, _L)] = 2 * N + jnp.bitwise_and(sdv, 16383)

    def _build_sidx(p):
        sdv = sd[p][...]
        dstv = jnp.bitwise_and(sdv, 16383)
        sidx[p][pl.ds(0, _L)] = dstv
        sidx[p][pl.ds(_L, _L)] = _NUMROWS + lax.shift_right_logical(dstv, 3)

    def _issue_gathers(p, ci):
        pltpu.async_copy(kvpq_hbm.at[gidx[p]], grow[p], sem_g)
        pltpu.async_copy(kvr_hbm.at[pl.ds(ci * _C, _C)], kvrb[p], sem_g)

    def _drain_gathers(p, ci):
        pltpu.make_async_copy(kvpq_hbm.at[gidx[p]], grow[p], sem_g).wait()
        pltpu.make_async_copy(kvr_hbm.at[pl.ds(ci * _C, _C)], kvrb[p],
                              sem_g).wait()

    def _compute(p):
        g = grow[p]
        kv_r = kvrb[p]
        sv = sval[p]

        def _edge(e, ecarry):
            del ecarry
            a = jnp.zeros((_L,), jnp.float32)
            qs = []
            for j8 in range(DH):
                sl = pl.ds(j8 * _L, _L)
                ke = g[e, sl] + kv_r[e, sl]
                pk = jnp.where(ke > 0, ke + 1.0, jnp.exp(ke))
                q16 = g[2 * _C + e, sl]
                qs.append(q16)
                a = a + jnp.abs(q16) * pk
            for j8 in range(DH):
                sv[e, pl.ds(j8 * _L, _L)] = a * (
                    g[_C + e, pl.ds(j8 * _L, _L)]
                    + kv_r[e, pl.ds(D + j8 * _L, _L)])
            zero = jnp.zeros((_L,), jnp.float32)
            for j8 in range(DH):
                sv[_C + e, pl.ds(j8 * _L, _L)] = jnp.where(
                    qs[j8] < 0, a, zero)
            return 0

        for e_ in range(_C):
            _edge(e_, 0)

    pltpu.sync_copy(sd_hbm.at[pl.ds(wid * _C, _C)], sd0)
    _build_gidx(0)
    _issue_gathers(0, wid)

    @pl.when(nj > 1)
    def _pro_idx1():
        pltpu.async_copy(sd_hbm.at[pl.ds((wid + _NW) * _C, _C)], sd1, sem_i)

    def _stage(j, p):
        q = 1 - p
        ci = j * _NW + wid
        _drain_gathers(p, ci)

        @pl.when(j >= 2)
        def _drain_scatter():
            pltpu.make_async_copy(sval[p], acc_sh.at[sidx[p]], sem_s).wait()

        _build_sidx(p)

        @pl.when(j + 1 < nj)
        def _prep_next():
            pltpu.make_async_copy(
                sd_hbm.at[pl.ds((ci + _NW) * _C, _C)], sd[q], sem_i).wait()
            _build_gidx(q)
            _issue_gathers(q, ci + _NW)

        @pl.when(j + 2 < nj)
        def _prefetch_idx():
            pltpu.async_copy(
                sd_hbm.at[pl.ds((ci + 2 * _NW) * _C, _C)], sd[p], sem_i)

        _compute(p)
        pltpu.async_copy(sval[p], acc_sh.at[sidx[p]], sem_s, add=True)

    def _pair(t, carry):
        for p in (0, 1):
            j = 2 * t + p

            @pl.when(j < nj)
            def _run():
                _stage(j, p)

        return carry

    lax.fori_loop(0, (nj + 1) // 2, _pair, 0)
    pltpu.make_async_copy(sval0, acc_sh.at[sidx0], sem_s).wait()
    pltpu.make_async_copy(sval1, acc_sh.at[sidx1], sem_s).wait()
    plsc.subcore_barrier()

    row0 = sid * _TROWS

    @pl.when(jnp.logical_not(last))
    def _out_main():
        pltpu.sync_copy(acc_sh.at[pl.ds(row0, _TROWS)],
                        num_out.at[cid, pl.ds(row0, _TROWS)])

    @pl.when(last)
    def _out_tail():
        pltpu.sync_copy(acc_sh.at[pl.ds(row0, _OUT_TAIL)],
                        num_out.at[cid, pl.ds(row0, _OUT_TAIL)])

    dbase = _NUMROWS + sid * 80
    ngrp = jnp.where(last, _OUT_TAIL // 64, _TROWS // 64)

    def _den_group(c, carry):
        pltpu.sync_copy(acc_sh.at[pl.ds(dbase + c * 8, 8)],
                        sval0.at[pl.ds(0, 8)])
        for qtr in range(4):
            def _unpack(i, carry2):
                n = qtr * _L + i
                denu[i, :] = sval0[n // DH, pl.ds((n % DH) * _L, _L)]
                return carry2

            lax.fori_loop(0, _L, _unpack, 0)
            pltpu.sync_copy(
                denu, den_out.at[cid, pl.ds(row0 + c * 64 + qtr * _L, _L)])
        return carry

    lax.fori_loop(0, ngrp, _den_group, 0)

    @pl.when(last)
    def _den_tail():
        pltpu.sync_copy(acc_sh.at[pl.ds(dbase + 48, 8)], sval0.at[pl.ds(0, 8)])

        def _unpack(i, carry2):
            denu[i, :] = sval0[i // DH, pl.ds((i % DH) * _L, _L)]
            return carry2

        lax.fori_loop(0, _L, _unpack, 0)
        pltpu.sync_copy(denu, den_out.at[cid, pl.ds(9984, _L)])


@functools.cache
def _build_edge_sc():
    return functools.partial(
        pl.kernel,
        out_type=(jax.ShapeDtypeStruct((_NC, N, D), jnp.float32),
                  jax.ShapeDtypeStruct((_NC, N, NH), jnp.float32)),
        mesh=plsc.VectorSubcoreMesh(core_axis_name="c", subcore_axis_name="s",
                                    num_cores=_NC, num_subcores=_NS),
        scratch_types=[
            pltpu.VMEM((_C,), jnp.int32),
            pltpu.VMEM((_C,), jnp.int32),
            pltpu.VMEM((3 * _C,), jnp.int32),
            pltpu.VMEM((3 * _C,), jnp.int32),
            pltpu.VMEM((2 * _C,), jnp.int32),
            pltpu.VMEM((2 * _C,), jnp.int32),
            pltpu.VMEM((3 * _C, D), jnp.float32),
            pltpu.VMEM((3 * _C, D), jnp.float32),
            pltpu.VMEM((_C, 2 * D), jnp.float32),
            pltpu.VMEM((_C, 2 * D), jnp.float32),
            pltpu.VMEM((2 * _C, D), jnp.float32),
            pltpu.VMEM((2 * _C, D), jnp.float32),
            pltpu.VMEM((_L, NH), jnp.float32),
            pltpu.VMEM_SHARED((_ACCROWS, D), jnp.float32),
            pltpu.SemaphoreType.DMA,
            pltpu.SemaphoreType.DMA,
            pltpu.SemaphoreType.DMA,
        ],
    )(_edge_sc_body)


_BN = 2000
_BE = 8000


def _rep_spec(shape):
    return pl.BlockSpec(shape, lambda i: (0,) * len(shape))


def kernel(x, r, edge_index, Wq, Wk, Wv, Wkr, Wvr, Wout, Wg, bg, Ws, bs,
           W1, b1, W2, b2, ln1_g, ln1_b, lnr_g, lnr_b, post_g, post_b,
           ffpre_g, ffpre_b, ffpost_g, ffpost_b):
    src = edge_index[0].astype(jnp.int32)
    dst = edge_index[1].astype(jnp.int32)

    def perm_cols(w):
        return w.reshape(D, NH, DH).transpose(0, 2, 1).reshape(D, NH * DH)

    wq_t = perm_cols(Wq)
    wk_t = perm_cols(Wk)
    wv_t = perm_cols(Wv)
    wkr_t = perm_cols(Wkr)
    wvr_t = perm_cols(Wvr)
    wout_p = Wout.reshape(NH, DH, D).transpose(1, 0, 2).reshape(NH * DH, D)
    wg1 = Wg[:D]
    wg2 = Wg[D:]

    def row(v):
        return v.reshape(1, -1)

    w_spec = _rep_spec((D, D))
    g_spec = _rep_spec((1, D))

    xn, pq, k, v = pl.pallas_call(
        _pre_node_body,
        grid=(N // _BN,),
        in_specs=[pl.BlockSpec((_BN, D), lambda i: (i, 0)),
                  w_spec, w_spec, w_spec, g_spec, g_spec],
        out_specs=[pl.BlockSpec((_BN, D), lambda i: (i, 0))] * 4,
        out_shape=[jax.ShapeDtypeStruct((N, D), jnp.float32)] * 4,
    )(x, wq_t, wk_t, wv_t, row(ln1_g), row(ln1_b))

    kvr = pl.pallas_call(
        _pre_edge_body,
        grid=(E // _BE,),
        in_specs=[pl.BlockSpec((_BE, D), lambda i: (i, 0)),
                  w_spec, w_spec, g_spec, g_spec],
        out_specs=pl.BlockSpec((_BE, 2 * D), lambda i: (i, 0)),
        out_shape=jax.ShapeDtypeStruct((E, 2 * D), jnp.float32),
    )(r, wkr_t, wvr_t, row(lnr_g), row(lnr_b))

    kvpq = jnp.concatenate([k, v, pq], axis=0)
    sd = jnp.bitwise_or(jnp.left_shift(src, 14), dst)
    num_p, den_p = _build_edge_sc()(sd, kvpq, kvr)

    y = pl.pallas_call(
        _post_body,
        grid=(N // _BN,),
        in_specs=[pl.BlockSpec((_BN, D), lambda i: (i, 0)),
                  pl.BlockSpec((_BN, D), lambda i: (i, 0)),
                  pl.BlockSpec((_NC, _BN, D), lambda i: (0, i, 0)),
                  pl.BlockSpec((_NC, _BN, NH), lambda i: (0, i, 0)),
                  w_spec, w_spec, w_spec, g_spec, w_spec, g_spec,
                  _rep_spec((D, F)), _rep_spec((1, F)),
                  _rep_spec((F, D)), g_spec,
                  g_spec, g_spec, g_spec, g_spec, g_spec, g_spec],
        out_specs=pl.BlockSpec((_BN, D), lambda i: (i, 0)),
        out_shape=jax.ShapeDtypeStruct((N, D), jnp.float32),
    )(x, xn, num_p, den_p, wout_p, wg1, wg2, row(bg), Ws, row(bs),
      W1, row(b1), W2, row(b2), row(post_g), row(post_b),
      row(ffpre_g), row(ffpre_b), row(ffpost_g), row(ffpost_b))

    return y
